# Initial kernel scaffold; baseline (speedup 1.0000x reference)
import jax
import jax.numpy as jnp
from jax.experimental import pallas as pl


def kernel(X, embL, embQ, bias):
    raise NotImplementedError("write your pallas kernel here")


# speedup vs baseline: 1.5059x; 1.5059x over previous
import functools

import jax
import jax.numpy as jnp
from jax import lax
from jax.experimental import pallas as pl
from jax.experimental.pallas import tpu as pltpu
from jax.experimental.pallas import tpu_sc as plsc

B = 16384
L = 50
D = 32
NUM_WORKERS = 32
SAMPLES_PER_WORKER = B // NUM_WORKERS
G = 16
NUM_GROUPS = SAMPLES_PER_WORKER // G


def _fm_body(x_hbm, embl_hbm, embq_hbm, bias_hbm, out_hbm,
             idx_v, q_v, el_v, bias_v, res_v, sem):
    wid = lax.axis_index("s") * 2 + lax.axis_index("c")
    iota = lax.broadcasted_iota(jnp.int32, (16,), 0)
    zero = jnp.zeros((16,), jnp.float32)

    pltpu.sync_copy(bias_hbm, bias_v)
    bias_vec = bias_v[...]

    def group_body(g, carry):
        s0 = wid * SAMPLES_PER_WORKER + g * G
        pltpu.sync_copy(x_hbm.at[pl.ds(s0, G)], idx_v)
        copies = []
        for s in range(G):
            copies.append(
                pltpu.async_copy(embq_hbm.at[idx_v.at[s]], q_v.at[s], sem))
            copies.append(
                pltpu.async_copy(embl_hbm.at[idx_v.at[s]], el_v.at[s], sem))
        for c in copies:
            c.wait()

        dvecs = [jnp.full((16,), d, jnp.int32) for d in range(D)]

        def l_body(l, lc):
            z, s2p, sl = lc
            lv = jnp.full((16,), l, jnp.int32)
            sl = sl + plsc.load_gather(el_v, [iota, lv])
            z = list(z)
            s2p = list(s2p)
            for d in range(D):
                a = plsc.load_gather(q_v, [iota, lv, dvecs[d]])
                z[d] = z[d] + a
                s2p[d % 4] = s2p[d % 4] + a * a
            return (tuple(z), tuple(s2p), sl)

        init = (tuple([zero] * D), (zero,) * 4, zero)
        z, s2p, sl = lax.fori_loop(0, L, l_body, init)

        s2 = (s2p[0] + s2p[1]) + (s2p[2] + s2p[3])
        z2p = [zero] * 4
        for d in range(D):
            z2p[d % 4] = z2p[d % 4] + z[d] * z[d]
        z2 = (z2p[0] + z2p[1]) + (z2p[2] + z2p[3])

        logit = sl + 0.5 * (s2 - z2) + bias_vec
        sig = 1.0 / (1.0 + jnp.exp(-logit))
        res_v[pl.ds(g * G, G)] = sig
        return carry

    lax.fori_loop(0, NUM_GROUPS, group_body, 0)
    pltpu.sync_copy(res_v,
                    out_hbm.at[pl.ds(wid * SAMPLES_PER_WORKER,
                                     SAMPLES_PER_WORKER)])


_fm_kernel = functools.partial(
    pl.kernel,
    mesh=plsc.VectorSubcoreMesh(core_axis_name="c", subcore_axis_name="s"),
    out_type=jax.ShapeDtypeStruct((B,), jnp.float32),
    compiler_params=pltpu.CompilerParams(
        needs_layout_passes=False, use_tc_tiling_on_sc=False),
    scratch_types=[
        pltpu.VMEM((G, L), jnp.int32),
        pltpu.VMEM((G, L, D), jnp.float32),
        pltpu.VMEM((G, L), jnp.float32),
        pltpu.VMEM((16,), jnp.float32),
        pltpu.VMEM((SAMPLES_PER_WORKER,), jnp.float32),
        pltpu.SemaphoreType.DMA,
    ],
)(_fm_body)


@jax.jit
def kernel(X, embL, embQ, bias):
    Xi = X.astype(jnp.int32)
    embL1 = embL.reshape((-1,))
    bias16 = jnp.broadcast_to(bias.reshape(()), (16,))
    return _fm_kernel(Xi, embL1, embQ, bias16)


# speedup vs baseline: 1.7722x; 1.1768x over previous
import functools

import jax
import jax.numpy as jnp
from jax import lax
from jax.experimental import pallas as pl
from jax.experimental.pallas import tpu as pltpu
from jax.experimental.pallas import tpu_sc as plsc

B = 16384
L = 50
D = 32
NUM_WORKERS = 32
SPW = B // NUM_WORKERS
G = 16
NUM_GROUPS = SPW // G


def _fm_body(x_hbm, embl_hbm, embq_hbm, bias_hbm, out_hbm,
             idx0, idx1, q0, q1, el0, el1, bias_v, res_v, sem0, sem1):
    wid = lax.axis_index("s") * 2 + lax.axis_index("c")
    iota = lax.broadcasted_iota(jnp.int32, (16,), 0)
    zeros_i = jnp.zeros((16,), jnp.int32)
    zero = jnp.zeros((16,), jnp.float32)

    pltpu.sync_copy(bias_hbm, bias_v)
    bias_vec = bias_v[...]

    slots = ((idx0, q0, el0, sem0), (idx1, q1, el1, sem1))

    def issue(g, slot):
        idx_v, q_v, el_v, sem = slot
        s0 = wid * SPW + g * G
        pltpu.sync_copy(x_hbm.at[pl.ds(s0, G)], idx_v)
        for s in range(G):
            pltpu.async_copy(embq_hbm.at[idx_v.at[s]], q_v.at[s], sem)
            pltpu.async_copy(embl_hbm.at[idx_v.at[s]], el_v.at[s], sem)

    def drain(slot):
        idx_v, q_v, el_v, sem = slot
        for s in range(G):
            pltpu.make_async_copy(
                embq_hbm.at[idx_v.at[s]], q_v.at[s], sem).wait()
            pltpu.make_async_copy(
                embl_hbm.at[idx_v.at[s]], el_v.at[s], sem).wait()

    def compute(g, slot):
        idx_v, q_v, el_v, sem = slot
        dvecs = [jnp.full((16,), d, jnp.int32) for d in range(D)]

        def l_body(l, lc):
            z, s2p, sl = lc
            lv = jnp.full((16,), l, jnp.int32)
            sl = sl + plsc.load_gather(el_v, [iota, lv])
            z = list(z)
            s2p = list(s2p)
            for d in range(D):
                a = plsc.load_gather(q_v, [iota, lv, dvecs[d]])
                z[d] = z[d] + a
                s2p[d % 4] = s2p[d % 4] + a * a
            return (tuple(z), tuple(s2p), sl)

        init = (tuple([zero] * D), (zero,) * 4, zero)
        z, s2p, sl = lax.fori_loop(0, L, l_body, init)

        s2 = (s2p[0] + s2p[1]) + (s2p[2] + s2p[3])
        z2p = [zero] * 4
        for d in range(D):
            z2p[d % 4] = z2p[d % 4] + z[d] * z[d]
        z2 = (z2p[0] + z2p[1]) + (z2p[2] + z2p[3])

        logit = sl + 0.5 * (s2 - z2) + bias_vec
        sig = 1.0 / (1.0 + jnp.exp(-logit))
        res_v[pl.ds(g * G, G)] = sig

    issue(0, slots[0])

    def t_body(t, carry):
        g0 = t * 2
        issue(g0 + 1, slots[1])
        drain(slots[0])
        compute(g0, slots[0])

        @pl.when(t < NUM_GROUPS // 2 - 1)
        def _():
            issue(g0 + 2, slots[0])

        drain(slots[1])
        compute(g0 + 1, slots[1])
        return carry

    lax.fori_loop(0, NUM_GROUPS // 2, t_body, 0)
    pltpu.sync_copy(res_v, out_hbm.at[pl.ds(wid * SPW, SPW)])


_fm_kernel = functools.partial(
    pl.kernel,
    mesh=plsc.VectorSubcoreMesh(core_axis_name="c", subcore_axis_name="s"),
    out_type=jax.ShapeDtypeStruct((B,), jnp.float32),
    compiler_params=pltpu.CompilerParams(
        needs_layout_passes=False, use_tc_tiling_on_sc=False),
    scratch_types=[
        pltpu.VMEM((G, L), jnp.int32),
        pltpu.VMEM((G, L), jnp.int32),
        pltpu.VMEM((G, L, D), jnp.float32),
        pltpu.VMEM((G, L, D), jnp.float32),
        pltpu.VMEM((G, L), jnp.float32),
        pltpu.VMEM((G, L), jnp.float32),
        pltpu.VMEM((16,), jnp.float32),
        pltpu.VMEM((SPW,), jnp.float32),
        pltpu.SemaphoreType.DMA,
        pltpu.SemaphoreType.DMA,
    ],
)(_fm_body)


@jax.jit
def kernel(X, embL, embQ, bias):
    Xi = X.astype(jnp.int32)
    embL1 = embL.reshape((-1,))
    bias16 = jnp.broadcast_to(bias.reshape(()), (16,))
    return _fm_kernel(Xi, embL1, embQ, bias16)
